# Initial kernel scaffold; baseline (speedup 1.0000x reference)
import jax
import jax.numpy as jnp
from jax.experimental import pallas as pl


def kernel(atom_fea, nbr_fea, nbr_fea_idx, crys_idx, W_emb1, b_emb1, W_emb2, b_emb2, fc_W, fc_b, bn1_g, bn1_b, bn2_g, bn2_b, head_W1, head_b1, head_W2, head_b2):
    raise NotImplementedError("write your pallas kernel here")


# speedup vs baseline: 2.4411x; 2.4411x over previous
import functools

import jax
import jax.numpy as jnp
from jax import lax
from jax.experimental import pallas as pl
from jax.experimental.pallas import tpu as pltpu
from jax.experimental.pallas import tpu_sc as plsc

_NC = 2
_NS = 16
_NW = _NC * _NS


def _sc_gather(table, idx, chunk):
    B, = idx.shape
    V, D = table.shape
    bpw = B // _NW
    n_chunks = bpw // chunk
    mesh = plsc.VectorSubcoreMesh(core_axis_name="c", subcore_axis_name="s")

    @functools.partial(
        pl.kernel,
        mesh=mesh,
        out_type=jax.ShapeDtypeStruct((B, D), table.dtype),
        compiler_params=pltpu.CompilerParams(use_tc_tiling_on_sc=False),
        scratch_types=[
            pltpu.VMEM((chunk,), jnp.int32),
            pltpu.VMEM((chunk, D), table.dtype),
            pltpu.SemaphoreType.DMA,
        ],
    )
    def k(table_hbm, idx_hbm, out_hbm, idx_v, rows_v, sem):
        wid = lax.axis_index("s") * _NC + lax.axis_index("c")
        for c in range(n_chunks):
            base = wid * bpw + c * chunk
            pltpu.sync_copy(idx_hbm.at[pl.ds(base, chunk)], idx_v)
            pltpu.async_copy(table_hbm.at[idx_v], rows_v, sem).wait()
            pltpu.sync_copy(rows_v, out_hbm.at[pl.ds(base, chunk)])

    return k(table, idx)


def _encoder(atom_fea, W1, b1, W2, b2):
    N, D0 = atom_fea.shape
    E = W1.shape[1]
    F = W2.shape[1]
    BA = 400
    grid = N // BA

    def body(a_r, w1_r, b1_r, w2_r, b2_r, o_r):
        h = jax.nn.softplus(
            jnp.dot(a_r[...], w1_r[...], preferred_element_type=jnp.float32)
            + b1_r[...])
        o_r[...] = jax.nn.softplus(
            jnp.dot(h, w2_r[...], preferred_element_type=jnp.float32)
            + b2_r[...])

    return pl.pallas_call(
        body,
        grid=(grid,),
        in_specs=[
            pl.BlockSpec((BA, D0), lambda i: (i, 0)),
            pl.BlockSpec((D0, E), lambda i: (0, 0)),
            pl.BlockSpec((1, E), lambda i: (0, 0)),
            pl.BlockSpec((E, F), lambda i: (0, 0)),
            pl.BlockSpec((1, F), lambda i: (0, 0)),
        ],
        out_specs=pl.BlockSpec((BA, F), lambda i: (i, 0)),
        out_shape=jax.ShapeDtypeStruct((N, F), jnp.float32),
    )(atom_fea, W1, b1.reshape(1, -1), W2, b2.reshape(1, -1))


_BA = 200


def _conv_stats(xg, nbr2, x, ws, wn, we, fcb):
    NM, F = xg.shape
    DN = nbr2.shape[1]
    N = x.shape[0]
    M = NM // N
    F2 = ws.shape[1]
    BA = _BA
    BE = BA * M
    grid = N // BA

    def body(xg_r, nbr_r, x_r, ws_r, wn_r, we_r, fcb_r, s_r, q_r):
        zs = jnp.dot(x_r[...], ws_r[...], preferred_element_type=jnp.float32) + fcb_r[...]
        g = jnp.dot(xg_r[...], wn_r[...], preferred_element_type=jnp.float32)
        g = g + jnp.dot(nbr_r[...], we_r[...], preferred_element_type=jnp.float32)
        g = g + jnp.broadcast_to(zs[:, None, :], (BA, M, F2)).reshape(BE, F2)

        @pl.when(pl.program_id(0) == 0)
        def _():
            s_r[...] = jnp.zeros_like(s_r)
            q_r[...] = jnp.zeros_like(q_r)

        s_r[...] += jnp.sum(g, axis=0, keepdims=True)
        q_r[...] += jnp.sum(g * g, axis=0, keepdims=True)

    return pl.pallas_call(
        body,
        grid=(grid,),
        in_specs=[
            pl.BlockSpec((BE, F), lambda i: (i, 0)),
            pl.BlockSpec((BE, DN), lambda i: (i, 0)),
            pl.BlockSpec((BA, F), lambda i: (i, 0)),
            pl.BlockSpec((F, F2), lambda i: (0, 0)),
            pl.BlockSpec((F, F2), lambda i: (0, 0)),
            pl.BlockSpec((DN, F2), lambda i: (0, 0)),
            pl.BlockSpec((1, F2), lambda i: (0, 0)),
        ],
        out_specs=[
            pl.BlockSpec((1, F2), lambda i: (0, 0)),
            pl.BlockSpec((1, F2), lambda i: (0, 0)),
        ],
        out_shape=[
            jax.ShapeDtypeStruct((1, F2), jnp.float32),
            jax.ShapeDtypeStruct((1, F2), jnp.float32),
        ],
    )(xg, nbr2, x, ws, wn, we, fcb)


def _conv_apply(xg, nbr2, x, ws, wn, we, fcb, s1, q1, g1, b1):
    NM, F = xg.shape
    DN = nbr2.shape[1]
    N = x.shape[0]
    M = NM // N
    F2 = ws.shape[1]
    BA = _BA
    BE = BA * M
    grid = N // BA
    inv_cnt = 1.0 / NM

    def body(xg_r, nbr_r, x_r, ws_r, wn_r, we_r, fcb_r, s1_r, q1_r, g1_r,
             b1_r, o_r, s2_r, q2_r):
        mu = s1_r[...] * inv_cnt
        var = q1_r[...] * inv_cnt - mu * mu
        sc = g1_r[...] * lax.rsqrt(var + 1e-5)
        sh = b1_r[...] - mu * sc

        zs = jnp.dot(x_r[...], ws_r[...], preferred_element_type=jnp.float32) + fcb_r[...]
        g = jnp.dot(xg_r[...], wn_r[...], preferred_element_type=jnp.float32)
        g = g + jnp.dot(nbr_r[...], we_r[...], preferred_element_type=jnp.float32)
        g = g + jnp.broadcast_to(zs[:, None, :], (BA, M, F2)).reshape(BE, F2)
        g = g * sc + sh

        f = g[:, :F]
        c = g[:, F:]
        act = jax.nn.sigmoid(f) * jax.nn.softplus(c)
        o = jnp.sum(act.reshape(BA, M, F), axis=1)
        o_r[...] = o

        @pl.when(pl.program_id(0) == 0)
        def _():
            s2_r[...] = jnp.zeros_like(s2_r)
            q2_r[...] = jnp.zeros_like(q2_r)

        s2_r[...] += jnp.sum(o, axis=0, keepdims=True)
        q2_r[...] += jnp.sum(o * o, axis=0, keepdims=True)

    return pl.pallas_call(
        body,
        grid=(grid,),
        in_specs=[
            pl.BlockSpec((BE, F), lambda i: (i, 0)),
            pl.BlockSpec((BE, DN), lambda i: (i, 0)),
            pl.BlockSpec((BA, F), lambda i: (i, 0)),
            pl.BlockSpec((F, F2), lambda i: (0, 0)),
            pl.BlockSpec((F, F2), lambda i: (0, 0)),
            pl.BlockSpec((DN, F2), lambda i: (0, 0)),
            pl.BlockSpec((1, F2), lambda i: (0, 0)),
            pl.BlockSpec((1, F2), lambda i: (0, 0)),
            pl.BlockSpec((1, F2), lambda i: (0, 0)),
            pl.BlockSpec((1, F2), lambda i: (0, 0)),
            pl.BlockSpec((1, F2), lambda i: (0, 0)),
        ],
        out_specs=[
            pl.BlockSpec((BA, F), lambda i: (i, 0)),
            pl.BlockSpec((1, F), lambda i: (0, 0)),
            pl.BlockSpec((1, F), lambda i: (0, 0)),
        ],
        out_shape=[
            jax.ShapeDtypeStruct((N, F), jnp.float32),
            jax.ShapeDtypeStruct((1, F), jnp.float32),
            jax.ShapeDtypeStruct((1, F), jnp.float32),
        ],
    )(xg, nbr2, x, ws, wn, we, fcb, s1, q1, g1, b1)


def _update(x, o, s2, q2, g2, b2):
    N, F = x.shape
    inv = 1.0 / N

    def body(x_r, o_r, s_r, q_r, g_r, b_r, out_r):
        mu = s_r[...] * inv
        var = q_r[...] * inv - mu * mu
        sc = g_r[...] * lax.rsqrt(var + 1e-5)
        sh = b_r[...] - mu * sc
        out_r[...] = jax.nn.softplus(x_r[...] + o_r[...] * sc + sh)

    return pl.pallas_call(
        body,
        out_shape=jax.ShapeDtypeStruct((N, F), jnp.float32),
    )(x, o, s2, q2, g2, b2)


def _pool_heads(rows, w1, b1, w2, b2, C, A):
    BP, F = rows.shape
    P, _, H = w1.shape

    def body(r_r, w1_r, b1_r, w2_r, b2_r, out_r):
        crys = jnp.mean(r_r[...][:C * A].reshape(C, A, F), axis=1)
        cols = []
        for p in range(P):
            h = jax.nn.softplus(
                jnp.dot(crys, w1_r[p], preferred_element_type=jnp.float32)
                + b1_r[p:p + 1, :])
            cols.append(jnp.sum(h * w2_r[p:p + 1, :], axis=1, keepdims=True)
                        + b2_r[0:1, p:p + 1])
        out_r[...] = jnp.concatenate(cols, axis=1)

    return pl.pallas_call(
        body,
        out_shape=jax.ShapeDtypeStruct((C, P), jnp.float32),
    )(rows, w1, b1, w2, b2)


def kernel(atom_fea, nbr_fea, nbr_fea_idx, crys_idx, W_emb1, b_emb1, W_emb2,
           b_emb2, fc_W, fc_b, bn1_g, bn1_b, bn2_g, bn2_b, head_W1, head_b1,
           head_W2, head_b2):
    N, D0 = atom_fea.shape
    _, M, DN = nbr_fea.shape
    F = W_emb2.shape[1]
    L = fc_W.shape[0]
    C, A = crys_idx.shape

    x = _encoder(atom_fea, W_emb1, b_emb1, W_emb2, b_emb2)

    nbr2 = nbr_fea.reshape(N * M, DN)
    nbr_flat = nbr_fea_idx.reshape(-1).astype(jnp.int32)

    for l in range(L):
        ws = fc_W[l, :F]
        wn = fc_W[l, F:2 * F]
        we = fc_W[l, 2 * F:]
        fcb = fc_b[l].reshape(1, -1)
        xg = _sc_gather(x, nbr_flat, 1000)
        s1, q1 = _conv_stats(xg, nbr2, x, ws, wn, we, fcb)
        o, s2, q2 = _conv_apply(xg, nbr2, x, ws, wn, we, fcb, s1, q1,
                                bn1_g[l].reshape(1, -1),
                                bn1_b[l].reshape(1, -1))
        x = _update(x, o, s2, q2, bn2_g[l].reshape(1, -1),
                    bn2_b[l].reshape(1, -1))

    pad = (-C * A) % (8 * _NW)
    cidx = jnp.concatenate([
        crys_idx.reshape(-1).astype(jnp.int32),
        jnp.zeros((pad,), jnp.int32),
    ])
    crows = _sc_gather(x, cidx, (C * A + pad) // _NW)
    out_t = _pool_heads(crows, head_W1, head_b1, head_W2,
                        head_b2.reshape(1, -1), C, A)
    return out_t.T


# speedup vs baseline: 2.7626x; 1.1317x over previous
import functools

import numpy as np

import jax
import jax.numpy as jnp
from jax import lax
from jax.experimental import pallas as pl
from jax.experimental.pallas import tpu as pltpu
from jax.experimental.pallas import tpu_sc as plsc

_NC = 2
_NS = 16
_NW = _NC * _NS


def _sc_gather(table, idx, chunk):
    B, = idx.shape
    V, D = table.shape
    bpw = B // _NW
    n_chunks = bpw // chunk
    mesh = plsc.VectorSubcoreMesh(core_axis_name="c", subcore_axis_name="s")

    @functools.partial(
        pl.kernel,
        mesh=mesh,
        out_type=jax.ShapeDtypeStruct((B, D), table.dtype),
        compiler_params=pltpu.CompilerParams(use_tc_tiling_on_sc=False),
        scratch_types=[
            pltpu.VMEM((chunk,), jnp.int32),
            pltpu.VMEM((chunk, D), table.dtype),
            pltpu.SemaphoreType.DMA,
        ],
    )
    def k(table_hbm, idx_hbm, out_hbm, idx_v, rows_v, sem):
        wid = lax.axis_index("s") * _NC + lax.axis_index("c")
        for c in range(n_chunks):
            base = wid * bpw + c * chunk
            pltpu.sync_copy(idx_hbm.at[pl.ds(base, chunk)], idx_v)
            pltpu.async_copy(table_hbm.at[idx_v], rows_v, sem).wait()
            pltpu.sync_copy(rows_v, out_hbm.at[pl.ds(base, chunk)])

    return k(table, idx)


def _encoder(atom_fea, W1, b1, W2, b2):
    N, D0 = atom_fea.shape
    E = W1.shape[1]
    F = W2.shape[1]
    BA = 400
    grid = N // BA

    def body(a_r, w1_r, b1_r, w2_r, b2_r, o_r):
        h = jax.nn.softplus(
            jnp.dot(a_r[...], w1_r[...], preferred_element_type=jnp.float32)
            + b1_r[...])
        o_r[...] = jax.nn.softplus(
            jnp.dot(h, w2_r[...], preferred_element_type=jnp.float32)
            + b2_r[...])

    return pl.pallas_call(
        body,
        grid=(grid,),
        in_specs=[
            pl.BlockSpec((BA, D0), lambda i: (i, 0)),
            pl.BlockSpec((D0, E), lambda i: (0, 0)),
            pl.BlockSpec((1, E), lambda i: (0, 0)),
            pl.BlockSpec((E, F), lambda i: (0, 0)),
            pl.BlockSpec((1, F), lambda i: (0, 0)),
        ],
        out_specs=pl.BlockSpec((BA, F), lambda i: (i, 0)),
        out_shape=jax.ShapeDtypeStruct((N, F), jnp.float32),
    )(atom_fea, W1, b1.reshape(1, -1), W2, b2.reshape(1, -1))


def _pre(x, ws, fcb):
    N, F = x.shape
    F2 = ws.shape[1]

    def body(x_r, w_r, b_r, o_r):
        o_r[...] = jnp.dot(x_r[...], w_r[...],
                           preferred_element_type=jnp.float32) + b_r[...]

    return pl.pallas_call(
        body,
        out_shape=jax.ShapeDtypeStruct((N, F2), jnp.float32),
    )(x, ws, fcb)


_BA = 200


def _edge_terms(xp_r, nte_r, nto_r, zs_r, wf_r, ws_r, we_r, BA, HM, F2):
    BE = BA * HM
    zsb = jnp.broadcast_to(zs_r[...][:, None, :], (BA, HM, F2)).reshape(BE, F2)
    xp = xp_r[...]
    dn = (((0,), (0,)), ((), ()))
    ef = lax.dot_general(nte_r[...], we_r[...], dn,
                         preferred_element_type=jnp.float32)
    es = lax.dot_general(nto_r[...], we_r[...], dn,
                         preferred_element_type=jnp.float32)
    gf = jnp.dot(xp, wf_r[...], preferred_element_type=jnp.float32) + ef + zsb
    gs = jnp.dot(xp, ws_r[...], preferred_element_type=jnp.float32) + es + zsb
    return gf, gs


def _conv_stats(xp, nte, nto, zs, wnf, wns, we):
    NP, F2 = xp.shape
    DN = nte.shape[0]
    N = zs.shape[0]
    HM = NP // N
    BA = _BA
    BE = BA * HM
    grid = N // BA

    def body(xp_r, nte_r, nto_r, zs_r, wf_r, ws_r, we_r, s_r, q_r):
        gf, gs = _edge_terms(xp_r, nte_r, nto_r, zs_r, wf_r, ws_r, we_r,
                             BA, HM, F2)

        @pl.when(pl.program_id(0) == 0)
        def _():
            s_r[...] = jnp.zeros_like(s_r)
            q_r[...] = jnp.zeros_like(q_r)

        s_r[...] += (jnp.sum(gf, axis=0, keepdims=True)
                     + jnp.sum(gs, axis=0, keepdims=True))
        q_r[...] += (jnp.sum(gf * gf, axis=0, keepdims=True)
                     + jnp.sum(gs * gs, axis=0, keepdims=True))

    return pl.pallas_call(
        body,
        grid=(grid,),
        in_specs=[
            pl.BlockSpec((BE, F2), lambda i: (i, 0)),
            pl.BlockSpec((DN, BE), lambda i: (0, i)),
            pl.BlockSpec((DN, BE), lambda i: (0, i)),
            pl.BlockSpec((BA, F2), lambda i: (i, 0)),
            pl.BlockSpec((F2, F2), lambda i: (0, 0)),
            pl.BlockSpec((F2, F2), lambda i: (0, 0)),
            pl.BlockSpec((DN, F2), lambda i: (0, 0)),
        ],
        out_specs=[
            pl.BlockSpec((1, F2), lambda i: (0, 0)),
            pl.BlockSpec((1, F2), lambda i: (0, 0)),
        ],
        out_shape=[
            jax.ShapeDtypeStruct((1, F2), jnp.float32),
            jax.ShapeDtypeStruct((1, F2), jnp.float32),
        ],
    )(xp, nte, nto, zs, wnf, wns, we)


def _conv_apply(xp, nte, nto, zs, wnf, wns, we, s1, q1, g1, b1):
    NP, F2 = xp.shape
    DN = nte.shape[0]
    N = zs.shape[0]
    HM = NP // N
    F = F2 // 2
    BA = _BA
    BE = BA * HM
    grid = N // BA
    inv_cnt = 1.0 / (N * HM * 2)

    def body(xp_r, nte_r, nto_r, zs_r, wf_r, ws_r, we_r, s1_r, q1_r, g1_r,
             b1_r, o_r, s2_r, q2_r):
        mu = s1_r[...] * inv_cnt
        var = q1_r[...] * inv_cnt - mu * mu
        sc = g1_r[...] * lax.rsqrt(var + 1e-5)
        sh = b1_r[...] - mu * sc

        gf, gs = _edge_terms(xp_r, nte_r, nto_r, zs_r, wf_r, ws_r, we_r,
                             BA, HM, F2)
        gf = gf * sc + sh
        gs = gs * sc + sh

        actf = jax.nn.sigmoid(gf[:, :F]) * jax.nn.softplus(gf[:, F:])
        acts = jax.nn.sigmoid(gs[:, :F]) * jax.nn.softplus(gs[:, F:])
        o = (jnp.sum(actf.reshape(BA, HM, F), axis=1)
             + jnp.sum(acts.reshape(BA, HM, F), axis=1))
        o_r[...] = o

        @pl.when(pl.program_id(0) == 0)
        def _():
            s2_r[...] = jnp.zeros_like(s2_r)
            q2_r[...] = jnp.zeros_like(q2_r)

        s2_r[...] += jnp.sum(o, axis=0, keepdims=True)
        q2_r[...] += jnp.sum(o * o, axis=0, keepdims=True)

    return pl.pallas_call(
        body,
        grid=(grid,),
        in_specs=[
            pl.BlockSpec((BE, F2), lambda i: (i, 0)),
            pl.BlockSpec((DN, BE), lambda i: (0, i)),
            pl.BlockSpec((DN, BE), lambda i: (0, i)),
            pl.BlockSpec((BA, F2), lambda i: (i, 0)),
            pl.BlockSpec((F2, F2), lambda i: (0, 0)),
            pl.BlockSpec((F2, F2), lambda i: (0, 0)),
            pl.BlockSpec((DN, F2), lambda i: (0, 0)),
            pl.BlockSpec((1, F2), lambda i: (0, 0)),
            pl.BlockSpec((1, F2), lambda i: (0, 0)),
            pl.BlockSpec((1, F2), lambda i: (0, 0)),
            pl.BlockSpec((1, F2), lambda i: (0, 0)),
        ],
        out_specs=[
            pl.BlockSpec((BA, F), lambda i: (i, 0)),
            pl.BlockSpec((1, F), lambda i: (0, 0)),
            pl.BlockSpec((1, F), lambda i: (0, 0)),
        ],
        out_shape=[
            jax.ShapeDtypeStruct((N, F), jnp.float32),
            jax.ShapeDtypeStruct((1, F), jnp.float32),
            jax.ShapeDtypeStruct((1, F), jnp.float32),
        ],
    )(xp, nte, nto, zs, wnf, wns, we, s1, q1, g1, b1)


def _update(x, o, s2, q2, g2, b2):
    N, F = x.shape
    inv = 1.0 / N

    def body(x_r, o_r, s_r, q_r, g_r, b_r, out_r):
        mu = s_r[...] * inv
        var = q_r[...] * inv - mu * mu
        sc = g_r[...] * lax.rsqrt(var + 1e-5)
        sh = b_r[...] - mu * sc
        out_r[...] = jax.nn.softplus(x_r[...] + o_r[...] * sc + sh)

    return pl.pallas_call(
        body,
        out_shape=jax.ShapeDtypeStruct((N, F), jnp.float32),
    )(x, o, s2, q2, g2, b2)


def _pool_heads(rows, w1, b1, w2, b2, C, A):
    BP, F = rows.shape
    P, _, H = w1.shape

    def body(r_r, w1_r, b1_r, w2_r, b2_r, out_r):
        crys = jnp.mean(r_r[...][:C * A].reshape(C, A, F), axis=1)
        cols = []
        for p in range(P):
            h = jax.nn.softplus(
                jnp.dot(crys, w1_r[p], preferred_element_type=jnp.float32)
                + b1_r[p:p + 1, :])
            cols.append(jnp.sum(h * w2_r[p:p + 1, :], axis=1, keepdims=True)
                        + b2_r[0:1, p:p + 1])
        out_r[...] = jnp.concatenate(cols, axis=1)

    return pl.pallas_call(
        body,
        out_shape=jax.ShapeDtypeStruct((C, P), jnp.float32),
    )(rows, w1, b1, w2, b2)


def kernel(atom_fea, nbr_fea, nbr_fea_idx, crys_idx, W_emb1, b_emb1, W_emb2,
           b_emb2, fc_W, fc_b, bn1_g, bn1_b, bn2_g, bn2_b, head_W1, head_b1,
           head_W2, head_b2):
    N, D0 = atom_fea.shape
    _, M, DN = nbr_fea.shape
    F = W_emb2.shape[1]
    L = fc_W.shape[0]
    C, A = crys_idx.shape
    HM = M // 2

    x = _encoder(atom_fea, W_emb1, b_emb1, W_emb2, b_emb2)

    perm = np.stack([np.arange(HM), np.arange(HM) + HM], axis=1).reshape(-1)
    idx_perm = nbr_fea_idx[:, perm].reshape(-1).astype(jnp.int32)

    nte = jnp.transpose(nbr_fea[:, :HM, :], (2, 0, 1)).reshape(DN, N * HM)
    nto = jnp.transpose(nbr_fea[:, HM:, :], (2, 0, 1)).reshape(DN, N * HM)

    for l in range(L):
        ws = fc_W[l, :F]
        wn = fc_W[l, F:2 * F]
        we = fc_W[l, 2 * F:]
        zero = jnp.zeros_like(wn)
        wnf = jnp.concatenate([wn, zero], axis=0)
        wns = jnp.concatenate([zero, wn], axis=0)
        zs = _pre(x, ws, fc_b[l].reshape(1, -1))
        xg = _sc_gather(x, idx_perm, 1000)
        xp = xg.reshape(N * HM, 2 * F)
        s1, q1 = _conv_stats(xp, nte, nto, zs, wnf, wns, we)
        o, s2, q2 = _conv_apply(xp, nte, nto, zs, wnf, wns, we, s1, q1,
                                bn1_g[l].reshape(1, -1),
                                bn1_b[l].reshape(1, -1))
        x = _update(x, o, s2, q2, bn2_g[l].reshape(1, -1),
                    bn2_b[l].reshape(1, -1))

    pad = (-C * A) % (8 * _NW)
    cidx = jnp.concatenate([
        crys_idx.reshape(-1).astype(jnp.int32),
        jnp.zeros((pad,), jnp.int32),
    ])
    crows = _sc_gather(x, cidx, (C * A + pad) // _NW)
    out_t = _pool_heads(crows, head_W1, head_b1, head_W2,
                        head_b2.reshape(1, -1), C, A)
    return out_t.T


# speedup vs baseline: 3.0954x; 1.1205x over previous
import functools

import numpy as np

import jax
import jax.numpy as jnp
from jax import lax
from jax.experimental import pallas as pl
from jax.experimental.pallas import tpu as pltpu
from jax.experimental.pallas import tpu_sc as plsc

_NC = 2
_NS = 16
_NW = _NC * _NS

_LOG2E = 1.4426950408889634
_LN2 = 0.6931471805599453


def _softplus(x):
    return jnp.log2(1.0 + jnp.exp2(x * _LOG2E)) * _LN2


def _sigmoid(x):
    return 1.0 / (1.0 + jnp.exp2(-x * _LOG2E))


def _sc_gather(table, idx, chunk):
    B, = idx.shape
    V, D = table.shape
    bpw = B // _NW
    n_chunks = bpw // chunk
    mesh = plsc.VectorSubcoreMesh(core_axis_name="c", subcore_axis_name="s")

    @functools.partial(
        pl.kernel,
        mesh=mesh,
        out_type=jax.ShapeDtypeStruct((B, D), table.dtype),
        compiler_params=pltpu.CompilerParams(use_tc_tiling_on_sc=False),
        scratch_types=[
            pltpu.VMEM((chunk,), jnp.int32),
            pltpu.VMEM((chunk, D), table.dtype),
            pltpu.SemaphoreType.DMA,
        ],
    )
    def k(table_hbm, idx_hbm, out_hbm, idx_v, rows_v, sem):
        wid = lax.axis_index("s") * _NC + lax.axis_index("c")
        for c in range(n_chunks):
            base = wid * bpw + c * chunk
            pltpu.sync_copy(idx_hbm.at[pl.ds(base, chunk)], idx_v)
            pltpu.async_copy(table_hbm.at[idx_v], rows_v, sem).wait()
            pltpu.sync_copy(rows_v, out_hbm.at[pl.ds(base, chunk)])

    return k(table, idx)


def _encoder(atom_fea, W1, b1, W2, b2):
    N, D0 = atom_fea.shape
    E = W1.shape[1]
    F = W2.shape[1]
    BA = 400
    grid = N // BA

    def body(a_r, w1_r, b1_r, w2_r, b2_r, o_r):
        h = _softplus(
            jnp.dot(a_r[...], w1_r[...], preferred_element_type=jnp.float32)
            + b1_r[...])
        o_r[...] = _softplus(
            jnp.dot(h, w2_r[...], preferred_element_type=jnp.float32)
            + b2_r[...])

    return pl.pallas_call(
        body,
        grid=(grid,),
        in_specs=[
            pl.BlockSpec((BA, D0), lambda i: (i, 0)),
            pl.BlockSpec((D0, E), lambda i: (0, 0)),
            pl.BlockSpec((1, E), lambda i: (0, 0)),
            pl.BlockSpec((E, F), lambda i: (0, 0)),
            pl.BlockSpec((1, F), lambda i: (0, 0)),
        ],
        out_specs=pl.BlockSpec((BA, F), lambda i: (i, 0)),
        out_shape=jax.ShapeDtypeStruct((N, F), jnp.float32),
    )(atom_fea, W1, b1.reshape(1, -1), W2, b2.reshape(1, -1))


def _pre(x, ws, fcb):
    N, F = x.shape
    F2 = ws.shape[1]

    def body(x_r, w_r, b_r, o_r):
        o_r[...] = jnp.dot(x_r[...], w_r[...],
                           preferred_element_type=jnp.float32) + b_r[...]

    return pl.pallas_call(
        body,
        out_shape=jax.ShapeDtypeStruct((N, F2), jnp.float32),
    )(x, ws, fcb)


_BA = 200


def _edge_terms(xp, nte, nto, zs, wf, ws, we, BA, HM, F2):
    BE = BA * HM
    zsb = jnp.broadcast_to(zs[:, None, :], (BA, HM, F2)).reshape(BE, F2)
    dn = (((0,), (0,)), ((), ()))
    ef = lax.dot_general(nte, we, dn, preferred_element_type=jnp.float32)
    es = lax.dot_general(nto, we, dn, preferred_element_type=jnp.float32)
    gf = jnp.dot(xp, wf, preferred_element_type=jnp.float32) + ef + zsb
    gs = jnp.dot(xp, ws, preferred_element_type=jnp.float32) + es + zsb
    return gf, gs


def _conv_stats(xp, nte, nto, zs, wnf, wns, we):
    NP, F2 = xp.shape
    DN = nte.shape[0]
    N = zs.shape[0]
    HM = NP // N
    BA = _BA
    BE = BA * HM
    grid = N // BA

    def body(xp_r, nte_r, nto_r, zs_r, wf_r, ws_r, we_r, s_r, q_r):
        gf, gs = _edge_terms(xp_r[...], nte_r[...], nto_r[...], zs_r[...],
                             wf_r[...], ws_r[...], we_r[...], BA, HM, F2)

        @pl.when(pl.program_id(0) == 0)
        def _():
            s_r[...] = jnp.zeros_like(s_r)
            q_r[...] = jnp.zeros_like(q_r)

        s_r[...] += (jnp.sum(gf, axis=0, keepdims=True)
                     + jnp.sum(gs, axis=0, keepdims=True))
        q_r[...] += (jnp.sum(gf * gf, axis=0, keepdims=True)
                     + jnp.sum(gs * gs, axis=0, keepdims=True))

    return pl.pallas_call(
        body,
        grid=(grid,),
        in_specs=[
            pl.BlockSpec((BE, F2), lambda i: (i, 0)),
            pl.BlockSpec((DN, BE), lambda i: (0, i)),
            pl.BlockSpec((DN, BE), lambda i: (0, i)),
            pl.BlockSpec((BA, F2), lambda i: (i, 0)),
            pl.BlockSpec((F2, F2), lambda i: (0, 0)),
            pl.BlockSpec((F2, F2), lambda i: (0, 0)),
            pl.BlockSpec((DN, F2), lambda i: (0, 0)),
        ],
        out_specs=[
            pl.BlockSpec((1, F2), lambda i: (0, 0)),
            pl.BlockSpec((1, F2), lambda i: (0, 0)),
        ],
        out_shape=[
            jax.ShapeDtypeStruct((1, F2), jnp.float32),
            jax.ShapeDtypeStruct((1, F2), jnp.float32),
        ],
    )(xp, nte, nto, zs, wnf, wns, we)


def _conv_apply(xp, nte, nto, zs, wnf, wns, we, s1, q1, g1, b1):
    NP, F2 = xp.shape
    DN = nte.shape[0]
    N = zs.shape[0]
    HM = NP // N
    F = F2 // 2
    BA = _BA
    BE = BA * HM
    grid = N // BA
    inv_cnt = 1.0 / (N * HM * 2)

    def body(xp_r, nte_r, nto_r, zs_r, wf_r, ws_r, we_r, s1_r, q1_r, g1_r,
             b1_r, o_r, s2_r, q2_r):
        mu = s1_r[...] * inv_cnt
        var = q1_r[...] * inv_cnt - mu * mu
        sc = g1_r[...] * lax.rsqrt(var + 1e-5)
        sh = b1_r[...] - mu * sc

        gf, gs = _edge_terms(xp_r[...], nte_r[...], nto_r[...],
                             zs_r[...] * sc + sh,
                             wf_r[...] * sc, ws_r[...] * sc, we_r[...] * sc,
                             BA, HM, F2)

        actf = _sigmoid(gf[:, :F]) * _softplus(gf[:, F:])
        acts = _sigmoid(gs[:, :F]) * _softplus(gs[:, F:])
        o = (jnp.sum(actf.reshape(BA, HM, F), axis=1)
             + jnp.sum(acts.reshape(BA, HM, F), axis=1))
        o_r[...] = o

        @pl.when(pl.program_id(0) == 0)
        def _():
            s2_r[...] = jnp.zeros_like(s2_r)
            q2_r[...] = jnp.zeros_like(q2_r)

        s2_r[...] += jnp.sum(o, axis=0, keepdims=True)
        q2_r[...] += jnp.sum(o * o, axis=0, keepdims=True)

    return pl.pallas_call(
        body,
        grid=(grid,),
        in_specs=[
            pl.BlockSpec((BE, F2), lambda i: (i, 0)),
            pl.BlockSpec((DN, BE), lambda i: (0, i)),
            pl.BlockSpec((DN, BE), lambda i: (0, i)),
            pl.BlockSpec((BA, F2), lambda i: (i, 0)),
            pl.BlockSpec((F2, F2), lambda i: (0, 0)),
            pl.BlockSpec((F2, F2), lambda i: (0, 0)),
            pl.BlockSpec((DN, F2), lambda i: (0, 0)),
            pl.BlockSpec((1, F2), lambda i: (0, 0)),
            pl.BlockSpec((1, F2), lambda i: (0, 0)),
            pl.BlockSpec((1, F2), lambda i: (0, 0)),
            pl.BlockSpec((1, F2), lambda i: (0, 0)),
        ],
        out_specs=[
            pl.BlockSpec((BA, F), lambda i: (i, 0)),
            pl.BlockSpec((1, F), lambda i: (0, 0)),
            pl.BlockSpec((1, F), lambda i: (0, 0)),
        ],
        out_shape=[
            jax.ShapeDtypeStruct((N, F), jnp.float32),
            jax.ShapeDtypeStruct((1, F), jnp.float32),
            jax.ShapeDtypeStruct((1, F), jnp.float32),
        ],
    )(xp, nte, nto, zs, wnf, wns, we, s1, q1, g1, b1)


def _update(x, o, s2, q2, g2, b2):
    N, F = x.shape
    inv = 1.0 / N

    def body(x_r, o_r, s_r, q_r, g_r, b_r, out_r):
        mu = s_r[...] * inv
        var = q_r[...] * inv - mu * mu
        sc = g_r[...] * lax.rsqrt(var + 1e-5)
        sh = b_r[...] - mu * sc
        out_r[...] = _softplus(x_r[...] + o_r[...] * sc + sh)

    return pl.pallas_call(
        body,
        out_shape=jax.ShapeDtypeStruct((N, F), jnp.float32),
    )(x, o, s2, q2, g2, b2)


def _pool_heads(rows, w1, b1, w2, b2, C, A):
    BP, F = rows.shape
    P, _, H = w1.shape

    def body(r_r, w1_r, b1_r, w2_r, b2_r, out_r):
        crys = jnp.mean(r_r[...][:C * A].reshape(C, A, F), axis=1)
        cols = []
        for p in range(P):
            h = _softplus(
                jnp.dot(crys, w1_r[p], preferred_element_type=jnp.float32)
                + b1_r[p:p + 1, :])
            cols.append(jnp.sum(h * w2_r[p:p + 1, :], axis=1, keepdims=True)
                        + b2_r[0:1, p:p + 1])
        out_r[...] = jnp.concatenate(cols, axis=1)

    return pl.pallas_call(
        body,
        out_shape=jax.ShapeDtypeStruct((C, P), jnp.float32),
    )(rows, w1, b1, w2, b2)


def kernel(atom_fea, nbr_fea, nbr_fea_idx, crys_idx, W_emb1, b_emb1, W_emb2,
           b_emb2, fc_W, fc_b, bn1_g, bn1_b, bn2_g, bn2_b, head_W1, head_b1,
           head_W2, head_b2):
    N, D0 = atom_fea.shape
    _, M, DN = nbr_fea.shape
    F = W_emb2.shape[1]
    L = fc_W.shape[0]
    C, A = crys_idx.shape
    HM = M // 2

    x = _encoder(atom_fea, W_emb1, b_emb1, W_emb2, b_emb2)

    perm = np.stack([np.arange(HM), np.arange(HM) + HM], axis=1).reshape(-1)
    idx_perm = nbr_fea_idx[:, perm].reshape(-1).astype(jnp.int32)

    nte = jnp.transpose(nbr_fea[:, :HM, :], (2, 0, 1)).reshape(DN, N * HM)
    nto = jnp.transpose(nbr_fea[:, HM:, :], (2, 0, 1)).reshape(DN, N * HM)

    for l in range(L):
        ws = fc_W[l, :F]
        wn = fc_W[l, F:2 * F]
        we = fc_W[l, 2 * F:]
        zero = jnp.zeros_like(wn)
        wnf = jnp.concatenate([wn, zero], axis=0)
        wns = jnp.concatenate([zero, wn], axis=0)
        zs = _pre(x, ws, fc_b[l].reshape(1, -1))
        xg = _sc_gather(x, idx_perm, 1000)
        xp = xg.reshape(N * HM, 2 * F)
        s1, q1 = _conv_stats(xp, nte, nto, zs, wnf, wns, we)
        o, s2, q2 = _conv_apply(xp, nte, nto, zs, wnf, wns, we, s1, q1,
                                bn1_g[l].reshape(1, -1),
                                bn1_b[l].reshape(1, -1))
        x = _update(x, o, s2, q2, bn2_g[l].reshape(1, -1),
                    bn2_b[l].reshape(1, -1))

    pad = (-C * A) % (8 * _NW)
    cidx = jnp.concatenate([
        crys_idx.reshape(-1).astype(jnp.int32),
        jnp.zeros((pad,), jnp.int32),
    ])
    crows = _sc_gather(x, cidx, (C * A + pad) // _NW)
    out_t = _pool_heads(crows, head_W1, head_b1, head_W2,
                        head_b2.reshape(1, -1), C, A)
    return out_t.T
